# baseline (speedup 1.0000x reference)
import functools

import jax
import jax.numpy as jnp
from jax import lax
from jax.experimental import pallas as pl
from jax.experimental.pallas import tpu as pltpu
from jax.experimental.pallas import tpu_sc as plsc

EMBED_DIM = 16
SEQ_LEN = 200
BATCH = 4096
B_TOTAL = BATCH * SEQ_LEN

NC = 2
NS = 16
NW = NC * NS
B_PER_W = B_TOTAL // NW
CHUNK = 1600
N_CHUNKS = B_PER_W // CHUNK
SEQS_PER_CHUNK = CHUNK // SEQ_LEN


def _positional_encoding_host(seq_len, embed_dim):
    pos = jnp.arange(seq_len, dtype=jnp.float32)[:, None]
    dim = jnp.arange(embed_dim, dtype=jnp.float32)[None, :]
    angle = pos / jnp.power(10000.0, 2.0 * dim / float(embed_dim))
    is_even = (jnp.arange(embed_dim)[None, :] % 2) == 0
    return jnp.where(is_even, jnp.sin(angle), jnp.cos(angle))


def _sc_body(table_hbm, idx_hbm, pe_hbm, out_hbm,
             idx_v, rows_v, pe_v, gsem):
    wid = lax.axis_index("s") * NC + lax.axis_index("c")
    base = wid * B_PER_W

    pltpu.sync_copy(pe_hbm, pe_v)

    def chunk_step(c, carry):
        cbase = base + c * CHUNK
        pltpu.sync_copy(idx_hbm.at[pl.ds(cbase, CHUNK)], idx_v)
        pltpu.async_copy(table_hbm.at[idx_v], rows_v, gsem).wait()

        def pe_step(j, _):
            pe_row = pe_v[j]
            for s in range(SEQS_PER_CHUNK):
                r = s * SEQ_LEN + j
                rows_v[r] = rows_v[r] + pe_row
            return 0

        lax.fori_loop(0, SEQ_LEN, pe_step, 0)
        pltpu.sync_copy(rows_v, out_hbm.at[pl.ds(cbase, CHUNK)])
        return carry

    lax.fori_loop(0, N_CHUNKS, chunk_step, 0)


@jax.jit
def _embed_lookup(x_flat, table, pe):
    mesh = plsc.VectorSubcoreMesh(core_axis_name="c", subcore_axis_name="s")
    return pl.kernel(
        _sc_body,
        out_type=jax.ShapeDtypeStruct((B_TOTAL, EMBED_DIM), jnp.float32),
        mesh=mesh,
        scratch_types=[
            pltpu.VMEM((CHUNK,), jnp.int32),
            pltpu.VMEM((CHUNK, EMBED_DIM), jnp.float32),
            pltpu.VMEM((SEQ_LEN, EMBED_DIM), jnp.float32),
            pltpu.SemaphoreType.DMA,
        ],
        compiler_params=pltpu.CompilerParams(use_tc_tiling_on_sc=False),
    )(table, x_flat, pe)


def kernel(x, table):
    pe = _positional_encoding_host(SEQ_LEN, EMBED_DIM)
    x_flat = x.reshape(B_TOTAL).astype(jnp.int32)
    out = _embed_lookup(x_flat, table, pe)
    return out.reshape(BATCH, SEQ_LEN, EMBED_DIM)


# speedup vs baseline: 1.0311x; 1.0311x over previous
import functools

import jax
import jax.numpy as jnp
from jax import lax
from jax.experimental import pallas as pl
from jax.experimental.pallas import tpu as pltpu
from jax.experimental.pallas import tpu_sc as plsc

EMBED_DIM = 16
SEQ_LEN = 200
BATCH = 4096
B_TOTAL = BATCH * SEQ_LEN

NC = 2
NS = 16
NW = NC * NS
B_PER_W = B_TOTAL // NW
CHUNK = 1600
N_CHUNKS = B_PER_W // CHUNK
SEQS_PER_CHUNK = CHUNK // SEQ_LEN


def _positional_encoding_host(seq_len, embed_dim):
    pos = jnp.arange(seq_len, dtype=jnp.float32)[:, None]
    dim = jnp.arange(embed_dim, dtype=jnp.float32)[None, :]
    angle = pos / jnp.power(10000.0, 2.0 * dim / float(embed_dim))
    is_even = (jnp.arange(embed_dim)[None, :] % 2) == 0
    return jnp.where(is_even, jnp.sin(angle), jnp.cos(angle))


def _sc_body(table_hbm, idx_hbm, pe_hbm, out_hbm,
             idx0, idx1, rows0, rows1, pe_v,
             gsem0, gsem1, osem0, osem1):
    wid = lax.axis_index("s") * NC + lax.axis_index("c")
    base = wid * B_PER_W

    idx_b = (idx0, idx1)
    rows_b = (rows0, rows1)
    gsem_b = (gsem0, gsem1)
    osem_b = (osem0, osem1)

    pltpu.sync_copy(pe_hbm, pe_v)

    def start_gather(c, b):
        cbase = base + c * CHUNK
        pltpu.sync_copy(idx_hbm.at[pl.ds(cbase, CHUNK)], idx_b[b])
        pltpu.async_copy(table_hbm.at[idx_b[b]], rows_b[b], gsem_b[b])

    def finish_chunk(c, b):
        pltpu.make_async_copy(table_hbm.at[idx_b[b]], rows_b[b],
                              gsem_b[b]).wait()

        def pe_step(j, _):
            pe_row = pe_v[j]
            for s in range(SEQS_PER_CHUNK):
                r = s * SEQ_LEN + j
                rows_b[b][r] = rows_b[b][r] + pe_row
            return 0

        lax.fori_loop(0, SEQ_LEN, pe_step, 0)
        cbase = base + c * CHUNK
        pltpu.async_copy(rows_b[b], out_hbm.at[pl.ds(cbase, CHUNK)],
                         osem_b[b])

    start_gather(0, 0)
    for c in range(N_CHUNKS):
        b = c % 2
        if c + 1 < N_CHUNKS:
            if c >= 1:
                cprev = c - 1
                pltpu.make_async_copy(
                    rows_b[1 - b],
                    out_hbm.at[pl.ds(base + cprev * CHUNK, CHUNK)],
                    osem_b[1 - b]).wait()
            start_gather(c + 1, 1 - b)
        finish_chunk(c, b)
    last = N_CHUNKS - 1
    for c in (last - 1, last):
        b = c % 2
        pltpu.make_async_copy(rows_b[b],
                              out_hbm.at[pl.ds(base + c * CHUNK, CHUNK)],
                              osem_b[b]).wait()


@jax.jit
def _embed_lookup(x_flat, table, pe):
    mesh = plsc.VectorSubcoreMesh(core_axis_name="c", subcore_axis_name="s")
    return pl.kernel(
        _sc_body,
        out_type=jax.ShapeDtypeStruct((B_TOTAL, EMBED_DIM), jnp.float32),
        mesh=mesh,
        scratch_types=[
            pltpu.VMEM((CHUNK,), jnp.int32),
            pltpu.VMEM((CHUNK,), jnp.int32),
            pltpu.VMEM((CHUNK, EMBED_DIM), jnp.float32),
            pltpu.VMEM((CHUNK, EMBED_DIM), jnp.float32),
            pltpu.VMEM((SEQ_LEN, EMBED_DIM), jnp.float32),
            pltpu.SemaphoreType.DMA,
            pltpu.SemaphoreType.DMA,
            pltpu.SemaphoreType.DMA,
            pltpu.SemaphoreType.DMA,
        ],
        compiler_params=pltpu.CompilerParams(use_tc_tiling_on_sc=False),
    )(table, x_flat, pe)


def kernel(x, table):
    pe = _positional_encoding_host(SEQ_LEN, EMBED_DIM)
    x_flat = x.reshape(B_TOTAL).astype(jnp.int32)
    out = _embed_lookup(x_flat, table, pe)
    return out.reshape(BATCH, SEQ_LEN, EMBED_DIM)


# speedup vs baseline: 1.2523x; 1.2146x over previous
import jax
import jax.numpy as jnp
from jax import lax
from jax.experimental import pallas as pl
from jax.experimental.pallas import tpu as pltpu
from jax.experimental.pallas import tpu_sc as plsc

EMBED_DIM = 16
SEQ_LEN = 200
BATCH = 4096

NC = 2
NS = 16
NW = NC * NS

ST = SEQ_LEN // 8
BT = BATCH // 128
N_BLOCKS = ST * BT
BLK_PER_W = N_BLOCKS // NW
ROWS_BLK = 8 * 128


def _positional_encoding_host(seq_len, embed_dim):
    pos = jnp.arange(seq_len, dtype=jnp.float32)[:, None]
    dim = jnp.arange(embed_dim, dtype=jnp.float32)[None, :]
    angle = pos / jnp.power(10000.0, 2.0 * dim / float(embed_dim))
    is_even = (jnp.arange(embed_dim)[None, :] % 2) == 0
    return jnp.where(is_even, jnp.sin(angle), jnp.cos(angle))


def _sc_body(table_hbm, xr_hbm, pe_hbm, out_hbm,
             idx_v, rows_v, obuf, pe_v, gsem, osem):
    w = lax.axis_index("s") * NC + lax.axis_index("c")
    pltpu.sync_copy(pe_hbm, pe_v)
    lane = lax.iota(jnp.int32, 16)

    def block_step(k, carry):
        blk = w * BLK_PER_W + k
        st = blk // BT
        bt = blk % BT
        pltpu.sync_copy(xr_hbm.at[st, bt], idx_v)
        pltpu.async_copy(table_hbm.at[idx_v], rows_v, gsem).wait()

        def d_step(t, _):
            dt = t // 8
            dr = t % 8
            col = jnp.full((16,), t, jnp.int32)
            for sr in range(8):
                s = st * 8 + sr
                pe_s = plsc.load_gather(pe_v, [jnp.full((16,), s, jnp.int32),
                                               col])
                for lg in range(8):
                    r0 = sr * 128 + lg * 16
                    vals = plsc.load_gather(rows_v, [r0 + lane, col])
                    obuf[sr, dt, dr, pl.ds(lg * 16, 16)] = vals + pe_s
            return 0

        lax.fori_loop(0, EMBED_DIM, d_step, 0)

        def s_store(sr, _):
            s = st * 8 + sr
            for dt in range(2):
                pltpu.sync_copy(obuf.at[sr, dt], out_hbm.at[s, dt, bt])
            return 0

        lax.fori_loop(0, 8, s_store, 0)
        return carry

    lax.fori_loop(0, BLK_PER_W, block_step, 0)


@jax.jit
def _embed_lookup(xr, table, pe):
    mesh = plsc.VectorSubcoreMesh(core_axis_name="c", subcore_axis_name="s")
    return pl.kernel(
        _sc_body,
        out_type=jax.ShapeDtypeStruct((SEQ_LEN, 2, BT, 8, 128),
                                      jnp.float32),
        mesh=mesh,
        scratch_types=[
            pltpu.VMEM((ROWS_BLK,), jnp.int32),
            pltpu.VMEM((ROWS_BLK, EMBED_DIM), jnp.float32),
            pltpu.VMEM((8, 2, 8, 128), jnp.float32),
            pltpu.VMEM((SEQ_LEN, EMBED_DIM), jnp.float32),
            pltpu.SemaphoreType.DMA,
            pltpu.SemaphoreType.DMA,
        ],
        compiler_params=pltpu.CompilerParams(use_tc_tiling_on_sc=False,
                                             needs_layout_passes=False),
    )(table, xr, pe)


def kernel(x, table):
    pe = _positional_encoding_host(SEQ_LEN, EMBED_DIM)
    xr = (x.astype(jnp.int32).T
          .reshape(ST, 8, BT, 128)
          .transpose(0, 2, 1, 3)
          .reshape(ST, BT, ROWS_BLK))
    out5 = _embed_lookup(xr, table, pe)
    return (out5.transpose(2, 4, 0, 1, 3)
            .reshape(BATCH, SEQ_LEN, EMBED_DIM))


# speedup vs baseline: 1.4140x; 1.1291x over previous
import jax
import jax.numpy as jnp
from jax import lax
from jax.experimental import pallas as pl
from jax.experimental.pallas import tpu as pltpu
from jax.experimental.pallas import tpu_sc as plsc

EMBED_DIM = 16
SEQ_LEN = 200
BATCH = 4096

NC = 2
NS = 16
NW = NC * NS

ST = SEQ_LEN // 8
BT = BATCH // 128
N_BLOCKS = ST * BT
BLK_PER_W = N_BLOCKS // NW
ROWS_BLK = 8 * 128


def _positional_encoding_host(seq_len, embed_dim):
    pos = jnp.arange(seq_len, dtype=jnp.float32)[:, None]
    dim = jnp.arange(embed_dim, dtype=jnp.float32)[None, :]
    angle = pos / jnp.power(10000.0, 2.0 * dim / float(embed_dim))
    is_even = (jnp.arange(embed_dim)[None, :] % 2) == 0
    return jnp.where(is_even, jnp.sin(angle), jnp.cos(angle))


def _sc_body(table_hbm, xr_hbm, pe_hbm, out_hbm,
             idx0, idx1, rows0, rows1, ob0, ob1, pe_v,
             isem0, isem1, gsem0, gsem1, osem0, osem1):
    w = lax.axis_index("s") * NC + lax.axis_index("c")
    idx_b = (idx0, idx1)
    rows_b = (rows0, rows1)
    ob_b = (ob0, ob1)
    isem_b = (isem0, isem1)
    gsem_b = (gsem0, gsem1)
    osem_b = (osem0, osem1)
    pltpu.sync_copy(pe_hbm, pe_v)
    lane = lax.iota(jnp.int32, 16)

    def tiles(k):
        blk = w * BLK_PER_W + k
        return blk // BT, blk % BT

    def idx_start(k, b):
        st, bt = tiles(k)
        pltpu.async_copy(xr_hbm.at[st, bt], idx_b[b], isem_b[b])

    def idx_wait(k, b):
        st, bt = tiles(k)
        pltpu.make_async_copy(xr_hbm.at[st, bt], idx_b[b],
                              isem_b[b]).wait()

    def gather_start(b):
        pltpu.async_copy(table_hbm.at[idx_b[b]], rows_b[b], gsem_b[b])

    def out_ref(k):
        st, bt = tiles(k)
        return out_hbm.at[pl.ds(st * 8, 8), :, bt]

    def store_wait(k, b):
        pltpu.make_async_copy(ob_b[b], out_ref(k), osem_b[b]).wait()

    idx_start(0, 0)
    idx_start(1, 1)
    idx_wait(0, 0)
    gather_start(0)

    def pair_step(kk, carry):
        for par in range(2):
            k = 2 * kk + par
            b = par
            pltpu.make_async_copy(table_hbm.at[idx_b[b]], rows_b[b],
                                  gsem_b[b]).wait()
            idx_wait(k + 1, 1 - b)
            gather_start(1 - b)

            @pl.when(k + 2 < BLK_PER_W)
            def _():
                idx_start(k + 2, b)

            @pl.when(k >= 2)
            def _():
                store_wait(k - 2, b)

            transpose_store_nowait(k, b)
        return carry

    def transpose_store_nowait(k, b):
        st, bt = tiles(k)
        rows_v = rows_b[b]
        obuf = ob_b[b]

        def d_step(t, _):
            dt = t // 8
            dr = t % 8
            col = jnp.full((16,), t, jnp.int32)
            for sr in range(8):
                s = st * 8 + sr
                pe_s = plsc.load_gather(pe_v, [jnp.full((16,), s, jnp.int32),
                                               col])
                for lg in range(8):
                    r0 = sr * 128 + lg * 16
                    vals = plsc.load_gather(rows_v, [r0 + lane, col])
                    obuf[sr, dt, dr, pl.ds(lg * 16, 16)] = vals + pe_s
            return 0

        lax.fori_loop(0, EMBED_DIM, d_step, 0)
        pltpu.async_copy(obuf, out_ref(k), osem_b[b])

    lax.fori_loop(0, (BLK_PER_W - 1) // 2, pair_step, 0)

    last = BLK_PER_W - 1
    pltpu.make_async_copy(table_hbm.at[idx_b[0]], rows_b[0],
                          gsem_b[0]).wait()
    store_wait(last - 2, 0)
    transpose_store_nowait(last, 0)
    store_wait(last - 1, 1)
    store_wait(last, 0)


@jax.jit
def _embed_lookup(xr, table, pe):
    mesh = plsc.VectorSubcoreMesh(core_axis_name="c", subcore_axis_name="s")
    return pl.kernel(
        _sc_body,
        out_type=jax.ShapeDtypeStruct((SEQ_LEN, 2, BT, 8, 128),
                                      jnp.float32),
        mesh=mesh,
        scratch_types=[
            pltpu.VMEM((ROWS_BLK,), jnp.int32),
            pltpu.VMEM((ROWS_BLK,), jnp.int32),
            pltpu.VMEM((ROWS_BLK, EMBED_DIM), jnp.float32),
            pltpu.VMEM((ROWS_BLK, EMBED_DIM), jnp.float32),
            pltpu.VMEM((8, 2, 8, 128), jnp.float32),
            pltpu.VMEM((8, 2, 8, 128), jnp.float32),
            pltpu.VMEM((SEQ_LEN, EMBED_DIM), jnp.float32),
            pltpu.SemaphoreType.DMA,
            pltpu.SemaphoreType.DMA,
            pltpu.SemaphoreType.DMA,
            pltpu.SemaphoreType.DMA,
            pltpu.SemaphoreType.DMA,
            pltpu.SemaphoreType.DMA,
        ],
        compiler_params=pltpu.CompilerParams(use_tc_tiling_on_sc=False,
                                             needs_layout_passes=False),
    )(table, xr, pe)


def kernel(x, table):
    pe = _positional_encoding_host(SEQ_LEN, EMBED_DIM)
    xr = (x.astype(jnp.int32).T
          .reshape(ST, 8, BT, 128)
          .transpose(0, 2, 1, 3)
          .reshape(ST, BT, ROWS_BLK))
    out5 = _embed_lookup(xr, table, pe)
    return (out5.transpose(2, 4, 0, 1, 3)
            .reshape(BATCH, SEQ_LEN, EMBED_DIM))
